# Initial kernel scaffold; baseline (speedup 1.0000x reference)
import jax
import jax.numpy as jnp
from jax.experimental import pallas as pl


def kernel(data, table):
    raise NotImplementedError("write your pallas kernel here")


# speedup vs baseline: 172.3406x; 172.3406x over previous
import functools

import jax
import jax.numpy as jnp
from jax import lax
from jax.experimental import pallas as pl
from jax.experimental.pallas import tpu as pltpu
from jax.experimental.pallas import tpu_sc as plsc

_NC = 2
_NS = 16
_NW = _NC * _NS
_L = 16
_BLK = 12800
_TABLE_PAD = 1024


@functools.partial(jax.jit, static_argnums=(2,))
def _lookup_call(table, flat_data, n_total):
    per_w = n_total // _NW
    nblk = per_w // _BLK
    vecs = _BLK // _L
    mesh = plsc.VectorSubcoreMesh(core_axis_name="c", subcore_axis_name="s")

    @functools.partial(
        pl.kernel,
        mesh=mesh,
        out_type=jax.ShapeDtypeStruct((n_total,), jnp.int32),
        scratch_types=[
            pltpu.VMEM((_TABLE_PAD,), jnp.int32),
            pltpu.VMEM((_BLK,), jnp.int32),
            pltpu.VMEM((_BLK,), jnp.int32),
        ],
        compiler_params=pltpu.CompilerParams(needs_layout_passes=False),
    )
    def lookup(table_hbm, data_hbm, out_hbm, table_v, idx_v, res_v):
        wid = lax.axis_index("s") * _NC + lax.axis_index("c")
        base = wid * per_w
        pltpu.sync_copy(table_hbm, table_v)

        def blk_body(b, carry):
            off = base + b * _BLK
            pltpu.sync_copy(data_hbm.at[pl.ds(off, _BLK)], idx_v)

            def vec_body(i, c2):
                sl = pl.ds(i * _L, _L)
                res_v[sl] = plsc.load_gather(table_v, [idx_v[sl]])
                return c2

            lax.fori_loop(0, vecs, vec_body, 0)
            pltpu.sync_copy(res_v, out_hbm.at[pl.ds(off, _BLK)])
            return carry

        lax.fori_loop(0, nblk, blk_body, 0)

    return lookup(table, flat_data)


def kernel(data, table):
    n = data.shape[0] * data.shape[1]
    flat = data.reshape((n,))
    table_p = jnp.zeros((_TABLE_PAD,), jnp.int32).at[: table.shape[0]].set(table)
    out = _lookup_call(table_p, flat, n)
    return out.reshape(data.shape)
